# baseline (speedup 1.0000x reference)
import functools

import jax
import jax.numpy as jnp
from jax import lax
from jax.experimental import pallas as pl
from jax.experimental.pallas import tpu as pltpu
from jax.experimental.pallas import tpu_sc as plsc

D = 32
N_NEI = 16384
N_PAIRS = 65536
N_TRAIN = 65536
SCALE = float(2.0 ** -16)

NC = 2
NS = 16
NW = NC * NS
L = 16

MAP_SIZE = 1 << 20
MROWS_PER_TILE = MAP_SIZE // NS
FRAG = MROWS_PER_TILE // 2

N_PROWS = 2 * N_NEI
PAIRS_PER_W = N_PAIRS // NW
CH = 512
NCHUNK = PAIRS_PER_W // CH
QN = CH // 128

_mesh = plsc.VectorSubcoreMesh(core_axis_name="c", subcore_axis_name="s")


@functools.partial(
    pl.kernel,
    out_type=(
        jax.ShapeDtypeStruct((N_PAIRS,), jnp.float32),
        jax.ShapeDtypeStruct((2 * NC * MAP_SIZE,), jnp.int32),
    ),
    mesh=_mesh,
    scratch_types=[
        pltpu.VMEM((FRAG,), jnp.int32),
        pltpu.VMEM((N_NEI,), jnp.int32),
        pltpu.VMEM((L,), jnp.int32),
        pltpu.VMEM((CH,), jnp.int32),
        pltpu.VMEM((CH,), jnp.int32),
        pltpu.VMEM((CH,), jnp.int32),
        pltpu.VMEM((CH,), jnp.int32),
        pltpu.VMEM((CH,), jnp.int32),
        pltpu.VMEM((CH,), jnp.int32),
        pltpu.VMEM((CH,), jnp.float32),
        pltpu.VMEM((CH,), jnp.float32),
        pltpu.VMEM((CH, D), jnp.float32),
        pltpu.VMEM((CH, D), jnp.float32),
        pltpu.VMEM((CH, D), jnp.float32),
        pltpu.VMEM((CH, D), jnp.float32),
        pltpu.VMEM((CH,), jnp.float32),
        pltpu.SemaphoreType.DMA,
        pltpu.SemaphoreType.DMA,
    ],
    compiler_params=pltpu.CompilerParams(needs_layout_passes=False,
                                         use_tc_tiling_on_sc=False),
)
def _unlearn_scores(ut_hbm, it_hbm, pext_hbm, nei_u_hbm, nei_i_hbm,
                    tu_hbm, ti_hbm, out_hbm, maps_hbm,
                    frag, nei, tmp16,
                    tu, ti, slot_u, slot_i, pidx_u, pidx_i, scale_u, scale_i,
                    gu, gi, pu, pi, sbuf, sem_a, sem_b):
    cc = lax.axis_index("c")
    s = lax.axis_index("s")
    wid = s * NC + cc
    lanes = lax.iota(jnp.int32, L)
    shift_idx = jnp.minimum(lanes + 1, L - 1)
    is_last_lane = lanes == (L - 1)
    scale = jnp.float32(SCALE)
    zero = jnp.float32(0.0)
    neg1 = jnp.full((L,), -1, jnp.int32)

    def build_half(which, h):
        base = s * MROWS_PER_TILE + h * FRAG

        def init_body(i, carry):
            frag[pl.ds(i * L, L)] = neg1
            return carry

        lax.fori_loop(0, FRAG // L, init_body, 0, unroll=8)

        def scan_body(g, carry):
            idx = nei[pl.ds(g * L, L)]
            key = (idx << 4) | lanes
            kvec = g * L + lanes
            skey, sval = plsc.sort_key_val(key, kvec)
            sidx = skey >> 4
            tmp16[...] = sidx
            nxt = plsc.load_gather(tmp16, [shift_idx])
            winner = (sidx != nxt) | is_last_lane
            local = sidx - base
            in_rng = plsc.bitcast(local, jnp.uint32) < jnp.uint32(FRAG)
            local_c = local & (FRAG - 1)
            plsc.store_scatter(frag, [local_c], sval, mask=winner & in_rng)
            return carry

        lax.fori_loop(0, N_NEI // L, scan_body, 0)
        off = (cc * 2 + which) * MAP_SIZE + base
        pltpu.sync_copy(frag, maps_hbm.at[pl.ds(off, FRAG)])

    pltpu.sync_copy(nei_u_hbm, nei)
    build_half(0, 0)
    build_half(0, 1)
    pltpu.sync_copy(nei_i_hbm, nei)
    build_half(1, 0)
    build_half(1, 1)
    plsc.subcore_barrier()

    def chunk_body(c, carry):
        pair_base = wid * PAIRS_PER_W + c * CH
        pltpu.sync_copy(tu_hbm.at[pl.ds(pair_base, CH)], tu)
        pltpu.sync_copy(ti_hbm.at[pl.ds(pair_base, CH)], ti)

        def midx_body(g, carry2):
            r = pl.ds(g * L, L)
            pidx_u[r] = tu[r] + (cc * 2) * MAP_SIZE
            pidx_i[r] = ti[r] + (cc * 2 + 1) * MAP_SIZE
            return carry2

        lax.fori_loop(0, CH // L, midx_body, 0, unroll=4)

        for q in range(QN):
            r = pl.ds(q * 128, 128)
            pltpu.async_copy(maps_hbm.at[pidx_u.at[r]], slot_u.at[r], sem_a)
            pltpu.async_copy(maps_hbm.at[pidx_i.at[r]], slot_i.at[r], sem_a)
            pltpu.async_copy(ut_hbm.at[tu.at[r]], gu.at[r, :], sem_b)
            pltpu.async_copy(it_hbm.at[ti.at[r]], gi.at[r, :], sem_b)
        for q in range(QN):
            r = pl.ds(q * 128, 128)
            pltpu.make_async_copy(maps_hbm.at[pidx_u.at[r]],
                                  slot_u.at[r], sem_a).wait()
            pltpu.make_async_copy(maps_hbm.at[pidx_i.at[r]],
                                  slot_i.at[r], sem_a).wait()

        def pidx_body(g, carry2):
            r = pl.ds(g * L, L)
            su = slot_u[r]
            si = slot_i[r]
            mu = su >= 0
            mi = si >= 0
            spread = (pair_base + g * L + lanes) & (N_NEI - 1)
            pidx_u[r] = jnp.where(mu, su, spread)
            pidx_i[r] = jnp.where(mi, si, spread) + N_NEI
            scale_u[r] = jnp.where(mu, scale, zero)
            scale_i[r] = jnp.where(mi, scale, zero)
            return carry2

        lax.fori_loop(0, CH // L, pidx_body, 0, unroll=4)

        for q in range(QN):
            r = pl.ds(q * 128, 128)
            pltpu.async_copy(pext_hbm.at[pidx_u.at[r]], pu.at[r, :], sem_a)
            pltpu.async_copy(pext_hbm.at[pidx_i.at[r]], pi.at[r, :], sem_a)
        for q in range(QN):
            r = pl.ds(q * 128, 128)
            pltpu.make_async_copy(ut_hbm.at[tu.at[r]], gu.at[r, :],
                                  sem_b).wait()
            pltpu.make_async_copy(it_hbm.at[ti.at[r]], gi.at[r, :],
                                  sem_b).wait()
            pltpu.make_async_copy(pext_hbm.at[pidx_u.at[r]], pu.at[r, :],
                                  sem_a).wait()
            pltpu.make_async_copy(pext_hbm.at[pidx_i.at[r]], pi.at[r, :],
                                  sem_a).wait()

        def dot_body(grp, carry2):
            rows = grp * L + lanes
            scu = scale_u[pl.ds(grp * L, L)]
            sci = scale_i[pl.ds(grp * L, L)]
            acc = jnp.zeros((L,), jnp.float32)
            for d in range(D):
                cold = jnp.full((L,), d, jnp.int32)
                au = plsc.load_gather(gu, [rows, cold])
                du = plsc.load_gather(pu, [rows, cold])
                ai = plsc.load_gather(gi, [rows, cold])
                di = plsc.load_gather(pi, [rows, cold])
                acc = acc + (au + scu * du) * (ai + sci * di)
            sbuf[pl.ds(grp * L, L)] = acc
            return carry2

        lax.fori_loop(0, CH // L, dot_body, 0)
        pltpu.sync_copy(sbuf, out_hbm.at[pl.ds(pair_base, CH)])
        return carry

    lax.fori_loop(0, NCHUNK, chunk_body, 0)


def kernel(user_table, item_table, p, nei_users, nei_items,
           train_users, train_items):
    scores, _ = _unlearn_scores(user_table, item_table, p.reshape(N_PROWS, D),
                                nei_users, nei_items,
                                train_users, train_items)
    return scores


# speedup vs baseline: 1.0999x; 1.0999x over previous
import functools

import jax
import jax.numpy as jnp
from jax import lax
from jax.experimental import pallas as pl
from jax.experimental.pallas import tpu as pltpu
from jax.experimental.pallas import tpu_sc as plsc

D = 32
N_NEI = 16384
N_PAIRS = 65536
N_TRAIN = 65536
SCALE = float(2.0 ** -16)

NC = 2
NS = 16
NW = NC * NS
L = 16

MAP_SIZE = 1 << 20
ROWS_PER_W = MAP_SIZE // NW

N_PROWS = 2 * N_NEI
N_DUMMY = 1024
PAIRS_PER_W = N_PAIRS // NW
CH = 512
NCHUNK = PAIRS_PER_W // CH
QN = CH // 128

_mesh = plsc.VectorSubcoreMesh(core_axis_name="c", subcore_axis_name="s")

@functools.partial(
    pl.kernel,
    out_type=(
        jax.ShapeDtypeStruct((MAP_SIZE,), jnp.int32),
        jax.ShapeDtypeStruct((MAP_SIZE,), jnp.int32),
    ),
    mesh=_mesh,
    scratch_types=[
        pltpu.VMEM((ROWS_PER_W,), jnp.int32),
        pltpu.VMEM((ROWS_PER_W,), jnp.int32),
        pltpu.VMEM((N_NEI,), jnp.int32),
        pltpu.VMEM((N_NEI,), jnp.int32),
        pltpu.VMEM((L,), jnp.int32),
    ],
    compiler_params=pltpu.CompilerParams(needs_layout_passes=False, use_tc_tiling_on_sc=False),
)
def _build_maps(nei_u_hbm, nei_i_hbm, map_u_hbm, map_i_hbm,
                frag_u, frag_i, nei_u, nei_i, tmp16):
    wid = lax.axis_index("s") * NC + lax.axis_index("c")
    base = wid * ROWS_PER_W

    neg1 = jnp.full((L,), -1, jnp.int32)

    def init_body(i, carry):
        frag_u[pl.ds(i * L, L)] = neg1
        frag_i[pl.ds(i * L, L)] = neg1
        return carry

    lax.fori_loop(0, ROWS_PER_W // L, init_body, 0, unroll=4)

    pltpu.sync_copy(nei_u_hbm, nei_u)
    pltpu.sync_copy(nei_i_hbm, nei_i)

    lanes = lax.iota(jnp.int32, L)
    shift_idx = jnp.minimum(lanes + 1, L - 1)
    is_last_lane = lanes == (L - 1)

    def scatter_group(frag, nei_ref, g):
        idx = nei_ref[pl.ds(g * L, L)]
        key = (idx << 4) | lanes
        kvec = g * L + lanes
        skey, sval = plsc.sort_key_val(key, kvec)
        sidx = skey >> 4
        tmp16[...] = sidx
        nxt = plsc.load_gather(tmp16, [shift_idx])
        winner = (sidx != nxt) | is_last_lane
        local = sidx - base
        in_range = plsc.bitcast(local, jnp.uint32) < jnp.uint32(ROWS_PER_W)
        local_c = local & (ROWS_PER_W - 1)
        plsc.store_scatter(frag, [local_c], sval, mask=winner & in_range)

    def body(g, carry):
        scatter_group(frag_u, nei_u, g)
        scatter_group(frag_i, nei_i, g)
        return carry

    lax.fori_loop(0, N_NEI // L, body, 0)

    pltpu.sync_copy(frag_u, map_u_hbm.at[pl.ds(base, ROWS_PER_W)])
    pltpu.sync_copy(frag_i, map_i_hbm.at[pl.ds(base, ROWS_PER_W)])


@functools.partial(
    pl.kernel,
    out_type=jax.ShapeDtypeStruct((N_PAIRS,), jnp.float32),
    mesh=_mesh,
    scratch_types=[
        pltpu.VMEM((CH,), jnp.int32),
        pltpu.VMEM((CH,), jnp.int32),
        pltpu.VMEM((CH,), jnp.int32),
        pltpu.VMEM((CH,), jnp.int32),
        pltpu.VMEM((CH,), jnp.int32),
        pltpu.VMEM((CH,), jnp.int32),
        pltpu.VMEM((CH,), jnp.float32),
        pltpu.VMEM((CH,), jnp.float32),
        pltpu.VMEM((CH, D), jnp.float32),
        pltpu.VMEM((CH, D), jnp.float32),
        pltpu.VMEM((CH, D), jnp.float32),
        pltpu.VMEM((CH, D), jnp.float32),
        pltpu.VMEM((CH,), jnp.float32),
        pltpu.SemaphoreType.DMA,
        pltpu.SemaphoreType.DMA,
    ],
    compiler_params=pltpu.CompilerParams(needs_layout_passes=False, use_tc_tiling_on_sc=False),
)
def _scores(ut_hbm, it_hbm, pext_hbm, map_u_hbm, map_i_hbm,
            tu_hbm, ti_hbm, out_hbm,
            tu, ti, slot_u, slot_i, pidx_u, pidx_i, scale_u, scale_i,
            gu, gi, pu, pi, sbuf, sem_a, sem_b):
    wid = lax.axis_index("s") * NC + lax.axis_index("c")
    lanes = lax.iota(jnp.int32, L)
    scale = jnp.float32(SCALE)
    zero = jnp.float32(0.0)

    def chunk_body(c, carry):
        pair_base = wid * PAIRS_PER_W + c * CH
        pltpu.sync_copy(tu_hbm.at[pl.ds(pair_base, CH)], tu)
        pltpu.sync_copy(ti_hbm.at[pl.ds(pair_base, CH)], ti)
        for q in range(QN):
            iu = tu.at[pl.ds(q * 128, 128)]
            ii = ti.at[pl.ds(q * 128, 128)]
            pltpu.async_copy(map_u_hbm.at[iu], slot_u.at[pl.ds(q * 128, 128)], sem_a)
            pltpu.async_copy(map_i_hbm.at[ii], slot_i.at[pl.ds(q * 128, 128)], sem_a)
            pltpu.async_copy(ut_hbm.at[iu], gu.at[pl.ds(q * 128, 128), :], sem_b)
            pltpu.async_copy(it_hbm.at[ii], gi.at[pl.ds(q * 128, 128), :], sem_b)
        for q in range(QN):
            iu = tu.at[pl.ds(q * 128, 128)]
            ii = ti.at[pl.ds(q * 128, 128)]
            pltpu.make_async_copy(map_u_hbm.at[iu], slot_u.at[pl.ds(q * 128, 128)], sem_a).wait()
            pltpu.make_async_copy(map_i_hbm.at[ii], slot_i.at[pl.ds(q * 128, 128)], sem_a).wait()

        def pidx_body(g, carry2):
            su = slot_u[pl.ds(g * L, L)]
            si = slot_i[pl.ds(g * L, L)]
            mu = su >= 0
            mi = si >= 0
            spread = (pair_base + g * L + lanes) & (N_NEI - 1)
            pidx_u[pl.ds(g * L, L)] = jnp.where(mu, su, spread)
            pidx_i[pl.ds(g * L, L)] = jnp.where(mi, si, spread) + N_NEI
            scale_u[pl.ds(g * L, L)] = jnp.where(mu, scale, zero)
            scale_i[pl.ds(g * L, L)] = jnp.where(mi, scale, zero)
            return carry2

        lax.fori_loop(0, CH // L, pidx_body, 0, unroll=4)

        for q in range(QN):
            pltpu.async_copy(pext_hbm.at[pidx_u.at[pl.ds(q * 128, 128)]],
                             pu.at[pl.ds(q * 128, 128), :], sem_a)
            pltpu.async_copy(pext_hbm.at[pidx_i.at[pl.ds(q * 128, 128)]],
                             pi.at[pl.ds(q * 128, 128), :], sem_a)
        for q in range(QN):
            pltpu.make_async_copy(ut_hbm.at[tu.at[pl.ds(q * 128, 128)]],
                                  gu.at[pl.ds(q * 128, 128), :], sem_b).wait()
            pltpu.make_async_copy(it_hbm.at[ti.at[pl.ds(q * 128, 128)]],
                                  gi.at[pl.ds(q * 128, 128), :], sem_b).wait()
            pltpu.make_async_copy(pext_hbm.at[pidx_u.at[pl.ds(q * 128, 128)]],
                                  pu.at[pl.ds(q * 128, 128), :], sem_a).wait()
            pltpu.make_async_copy(pext_hbm.at[pidx_i.at[pl.ds(q * 128, 128)]],
                                  pi.at[pl.ds(q * 128, 128), :], sem_a).wait()

        def dot_body(grp, carry2):
            rows = grp * L + lanes
            scu = scale_u[pl.ds(grp * L, L)]
            sci = scale_i[pl.ds(grp * L, L)]
            acc = jnp.zeros((L,), jnp.float32)
            for d in range(D):
                cold = jnp.full((L,), d, jnp.int32)
                au = plsc.load_gather(gu, [rows, cold])
                du = plsc.load_gather(pu, [rows, cold])
                ai = plsc.load_gather(gi, [rows, cold])
                di = plsc.load_gather(pi, [rows, cold])
                acc = acc + (au + scu * du) * (ai + sci * di)
            sbuf[pl.ds(grp * L, L)] = acc
            return carry2

        lax.fori_loop(0, CH // L, dot_body, 0)
        pltpu.sync_copy(sbuf, out_hbm.at[pl.ds(pair_base, CH)])
        return carry

    lax.fori_loop(0, NCHUNK, chunk_body, 0)


def kernel(user_table, item_table, p, nei_users, nei_items,
           train_users, train_items):
    map_u, map_i = _build_maps(nei_users, nei_items)
    return _scores(user_table, item_table, p.reshape(N_PROWS, D),
                   map_u, map_i, train_users, train_items)
